# Initial kernel scaffold; baseline (speedup 1.0000x reference)
import functools

import jax
import jax.numpy as jnp
from jax import lax
from jax.experimental import pallas as pl
from jax.experimental.pallas import tpu as pltpu
from jax.experimental.pallas import tpu_sc as plsc

N = 10000
E = 320000
D = 128
NC = 2
NS = 16
NW = NC * NS
K = 80
BPW = E // NW // K
RPS = N // NS
DW = 8

_mesh = plsc.VectorSubcoreMesh(core_axis_name="c", subcore_axis_name="s")


@functools.partial(
    pl.kernel,
    out_type=jax.ShapeDtypeStruct((NC, N, DW), jnp.float32),
    mesh=_mesh,
    scratch_types=[
        pltpu.VMEM_SHARED((N, DW), jnp.float32),
        pltpu.VMEM((BPW, K), jnp.int32),
        pltpu.VMEM((K, DW), jnp.float32),
    ],
)
def _sc_degree(col_hbm, zeros_hbm, ones_hbm, out_hbm, acc, cidx, ones):
    cid = lax.axis_index("c")
    sid = lax.axis_index("s")
    wid = sid * NC + cid
    pltpu.sync_copy(zeros_hbm.at[pl.ds(sid * RPS, RPS)],
                    acc.at[pl.ds(sid * RPS, RPS)])
    pltpu.sync_copy(col_hbm.at[pl.ds(wid * BPW, BPW)], cidx)
    pltpu.sync_copy(ones_hbm, ones)
    plsc.subcore_barrier()

    def body(j, carry):
        pltpu.sync_copy(ones, acc.at[cidx.at[j]], add=True)
        return carry

    lax.fori_loop(0, BPW, body, 0)
    plsc.subcore_barrier()
    pltpu.sync_copy(acc.at[pl.ds(sid * RPS, RPS)],
                    out_hbm.at[cid].at[pl.ds(sid * RPS, RPS)])


@functools.partial(
    pl.kernel,
    out_type=jax.ShapeDtypeStruct((NC, N, D), jnp.float32),
    mesh=_mesh,
    scratch_types=[
        pltpu.VMEM_SHARED((N, D), jnp.float32),
        pltpu.VMEM((BPW, K), jnp.int32),
        pltpu.VMEM((BPW, K), jnp.int32),
        pltpu.VMEM((K, D), jnp.float32),
        pltpu.SemaphoreType.DMA,
    ],
)
def _sc_aggregate(hs_hbm, row_hbm, col_hbm, zeros_hbm, out_hbm,
                  acc, ridx, cidx, rows, sem):
    cid = lax.axis_index("c")
    sid = lax.axis_index("s")
    wid = sid * NC + cid
    pltpu.sync_copy(zeros_hbm.at[pl.ds(sid * RPS, RPS)],
                    acc.at[pl.ds(sid * RPS, RPS)])
    pltpu.sync_copy(row_hbm.at[pl.ds(wid * BPW, BPW)], ridx)
    pltpu.sync_copy(col_hbm.at[pl.ds(wid * BPW, BPW)], cidx)
    plsc.subcore_barrier()

    def body(j, carry):
        pltpu.async_copy(hs_hbm.at[ridx.at[j]], rows, sem).wait()
        pltpu.sync_copy(rows, acc.at[cidx.at[j]], add=True)
        return carry

    lax.fori_loop(0, BPW, body, 0)
    plsc.subcore_barrier()
    pltpu.sync_copy(acc.at[pl.ds(sid * RPS, RPS)],
                    out_hbm.at[cid].at[pl.ds(sid * RPS, RPS)])


RB = 400


def _dinv_block(deg_ref):
    d = deg_ref[0][:, 0:1] + deg_ref[1][:, 0:1]
    return lax.rsqrt(d + 1.0)


def _tc_first(x, W1, deg):
    def body(x_ref, w_ref, deg_ref, o_ref):
        mm = jnp.dot(x_ref[...], w_ref[...], preferred_element_type=jnp.float32)
        o_ref[...] = mm * _dinv_block(deg_ref)

    return pl.pallas_call(
        body,
        grid=(N // RB,),
        in_specs=[
            pl.BlockSpec((RB, D), lambda i: (i, 0)),
            pl.BlockSpec((D, D), lambda i: (0, 0)),
            pl.BlockSpec((NC, RB, DW), lambda i: (0, i, 0)),
        ],
        out_specs=pl.BlockSpec((RB, D), lambda i: (i, 0)),
        out_shape=jax.ShapeDtypeStruct((N, D), jnp.float32),
    )(x, W1, deg)


def _tc_mid(A, hs, b, W2, deg):
    def body(a_ref, hs_ref, b_ref, w_ref, deg_ref, o_ref):
        dinv = _dinv_block(deg_ref)
        h = (a_ref[0] + a_ref[1] + hs_ref[...]) * dinv + b_ref[...]
        h = jnp.maximum(h, 0.0)
        mm = jnp.dot(h, w_ref[...], preferred_element_type=jnp.float32)
        o_ref[...] = mm * dinv

    return pl.pallas_call(
        body,
        grid=(N // RB,),
        in_specs=[
            pl.BlockSpec((NC, RB, D), lambda i: (0, i, 0)),
            pl.BlockSpec((RB, D), lambda i: (i, 0)),
            pl.BlockSpec((1, D), lambda i: (0, 0)),
            pl.BlockSpec((D, D), lambda i: (0, 0)),
            pl.BlockSpec((NC, RB, DW), lambda i: (0, i, 0)),
        ],
        out_specs=pl.BlockSpec((RB, D), lambda i: (i, 0)),
        out_shape=jax.ShapeDtypeStruct((N, D), jnp.float32),
    )(A, hs, b, W2, deg)


def _tc_final(A, hs, b, deg):
    def body(a_ref, hs_ref, b_ref, deg_ref, o_ref):
        dinv = _dinv_block(deg_ref)
        h = (a_ref[0] + a_ref[1] + hs_ref[...]) * dinv + b_ref[...]
        o_ref[...] = jnp.maximum(h, 0.0)

    return pl.pallas_call(
        body,
        grid=(N // RB,),
        in_specs=[
            pl.BlockSpec((NC, RB, D), lambda i: (0, i, 0)),
            pl.BlockSpec((RB, D), lambda i: (i, 0)),
            pl.BlockSpec((1, D), lambda i: (0, 0)),
            pl.BlockSpec((NC, RB, DW), lambda i: (0, i, 0)),
        ],
        out_specs=pl.BlockSpec((RB, D), lambda i: (i, 0)),
        out_shape=jax.ShapeDtypeStruct((N, D), jnp.float32),
    )(A, hs, b, deg)


def kernel(x, edge_index, W1, b1, W2, b2):
    row = edge_index[0].astype(jnp.int32).reshape(NW * BPW, K)
    col = edge_index[1].astype(jnp.int32).reshape(NW * BPW, K)
    zeros = jnp.zeros((N, D), jnp.float32)
    zeros8 = jnp.zeros((N, DW), jnp.float32)
    ones8 = jnp.ones((K, DW), jnp.float32)
    b1r = b1.reshape(1, D)
    b2r = b2.reshape(1, D)

    deg = _sc_degree(col, zeros8, ones8)
    hs1 = _tc_first(x, W1, deg)
    A1 = _sc_aggregate(hs1, row, col, zeros)
    hs2 = _tc_mid(A1, hs1, b1r, W2, deg)
    A2 = _sc_aggregate(hs2, row, col, zeros)
    return _tc_final(A2, hs2, b2r, deg)


# speedup vs baseline: 17.1531x; 17.1531x over previous
import functools

import jax
import jax.numpy as jnp
from jax import lax
from jax.experimental import pallas as pl
from jax.experimental.pallas import tpu as pltpu
from jax.experimental.pallas import tpu_sc as plsc

N = 10000
E = 320000
D = 128
NC = 2
NS = 16
NW = NC * NS
K = 80
BPW = E // NW // K
NP = 10240
RPS = NP // NS

_mesh = plsc.VectorSubcoreMesh(core_axis_name="c", subcore_axis_name="s",
                               num_cores=NC, num_subcores=NS)


@functools.partial(
    pl.kernel,
    out_type=jax.ShapeDtypeStruct((NC, NP, D), jnp.float32),
    mesh=_mesh,
    scratch_types=[
        pltpu.VMEM_SHARED((NP, D), jnp.float32),
        pltpu.VMEM((BPW, K), jnp.int32),
        pltpu.VMEM((K, D), jnp.float32),
    ],
)
def _sc_degree(col_hbm, zeros_hbm, ones_hbm, out_hbm, acc, cidx, ones):
    cid = lax.axis_index("c")
    sid = lax.axis_index("s")
    wid = sid * NC + cid
    pltpu.sync_copy(zeros_hbm.at[pl.ds(sid * RPS, RPS)],
                    acc.at[pl.ds(sid * RPS, RPS)])
    pltpu.sync_copy(col_hbm.at[wid], cidx)
    pltpu.sync_copy(ones_hbm, ones)
    plsc.subcore_barrier()

    def body(j, carry):
        pltpu.sync_copy(ones, acc.at[cidx.at[j]], add=True)
        return carry

    lax.fori_loop(0, BPW, body, 0)
    plsc.subcore_barrier()
    pltpu.sync_copy(acc.at[pl.ds(sid * RPS, RPS)],
                    out_hbm.at[cid].at[pl.ds(sid * RPS, RPS)])


@functools.partial(
    pl.kernel,
    out_type=jax.ShapeDtypeStruct((NC, NP, D), jnp.float32),
    mesh=_mesh,
    scratch_types=[
        pltpu.VMEM_SHARED((NP, D), jnp.float32),
        pltpu.VMEM((BPW, K), jnp.int32),
        pltpu.VMEM((BPW, K), jnp.int32),
        pltpu.VMEM((K, D), jnp.float32),
        pltpu.SemaphoreType.DMA,
    ],
)
def _sc_aggregate(hs_hbm, row_hbm, col_hbm, zeros_hbm, out_hbm,
                  acc, ridx, cidx, rows, sem):
    cid = lax.axis_index("c")
    sid = lax.axis_index("s")
    wid = sid * NC + cid
    pltpu.sync_copy(zeros_hbm.at[pl.ds(sid * RPS, RPS)],
                    acc.at[pl.ds(sid * RPS, RPS)])
    pltpu.sync_copy(row_hbm.at[wid], ridx)
    pltpu.sync_copy(col_hbm.at[wid], cidx)
    plsc.subcore_barrier()

    def body(j, carry):
        pltpu.async_copy(hs_hbm.at[ridx.at[j]], rows, sem).wait()
        pltpu.sync_copy(rows, acc.at[cidx.at[j]], add=True)
        return carry

    lax.fori_loop(0, BPW, body, 0)
    plsc.subcore_barrier()
    pltpu.sync_copy(acc.at[pl.ds(sid * RPS, RPS)],
                    out_hbm.at[cid].at[pl.ds(sid * RPS, RPS)])


RB = 400


def _dinv_block(deg_ref):
    d = deg_ref[0][:, 0:1] + deg_ref[1][:, 0:1]
    return lax.rsqrt(d + 1.0)


def _tc_first(x, W1, deg):
    def body(x_ref, w_ref, deg_ref, o_ref):
        mm = jnp.dot(x_ref[...], w_ref[...], preferred_element_type=jnp.float32)
        o_ref[...] = mm * _dinv_block(deg_ref)

    return pl.pallas_call(
        body,
        grid=(N // RB,),
        in_specs=[
            pl.BlockSpec((RB, D), lambda i: (i, 0)),
            pl.BlockSpec((D, D), lambda i: (0, 0)),
            pl.BlockSpec((NC, RB, D), lambda i: (0, i, 0)),
        ],
        out_specs=pl.BlockSpec((RB, D), lambda i: (i, 0)),
        out_shape=jax.ShapeDtypeStruct((N, D), jnp.float32),
    )(x, W1, deg)


def _tc_mid(A, hs, b, W2, deg):
    def body(a_ref, hs_ref, b_ref, w_ref, deg_ref, o_ref):
        dinv = _dinv_block(deg_ref)
        h = (a_ref[0] + a_ref[1] + hs_ref[...]) * dinv + b_ref[...]
        h = jnp.maximum(h, 0.0)
        mm = jnp.dot(h, w_ref[...], preferred_element_type=jnp.float32)
        o_ref[...] = mm * dinv

    return pl.pallas_call(
        body,
        grid=(N // RB,),
        in_specs=[
            pl.BlockSpec((NC, RB, D), lambda i: (0, i, 0)),
            pl.BlockSpec((RB, D), lambda i: (i, 0)),
            pl.BlockSpec((1, D), lambda i: (0, 0)),
            pl.BlockSpec((D, D), lambda i: (0, 0)),
            pl.BlockSpec((NC, RB, D), lambda i: (0, i, 0)),
        ],
        out_specs=pl.BlockSpec((RB, D), lambda i: (i, 0)),
        out_shape=jax.ShapeDtypeStruct((N, D), jnp.float32),
    )(A, hs, b, W2, deg)


def _tc_final(A, hs, b, deg):
    def body(a_ref, hs_ref, b_ref, deg_ref, o_ref):
        dinv = _dinv_block(deg_ref)
        h = (a_ref[0] + a_ref[1] + hs_ref[...]) * dinv + b_ref[...]
        o_ref[...] = jnp.maximum(h, 0.0)

    return pl.pallas_call(
        body,
        grid=(N // RB,),
        in_specs=[
            pl.BlockSpec((NC, RB, D), lambda i: (0, i, 0)),
            pl.BlockSpec((RB, D), lambda i: (i, 0)),
            pl.BlockSpec((1, D), lambda i: (0, 0)),
            pl.BlockSpec((NC, RB, D), lambda i: (0, i, 0)),
        ],
        out_specs=pl.BlockSpec((RB, D), lambda i: (i, 0)),
        out_shape=jax.ShapeDtypeStruct((N, D), jnp.float32),
    )(A, hs, b, deg)


def kernel(x, edge_index, W1, b1, W2, b2):
    row = edge_index[0].astype(jnp.int32).reshape(NW, BPW, K)
    col = edge_index[1].astype(jnp.int32).reshape(NW, BPW, K)
    zeros = jnp.zeros((NP, D), jnp.float32)
    onesD = jnp.ones((K, D), jnp.float32)
    b1r = b1.reshape(1, D)
    b2r = b2.reshape(1, D)

    deg = _sc_degree(col, zeros, onesD)
    hs1 = _tc_first(x, W1, deg)
    A1 = _sc_aggregate(hs1, row, col, zeros)
    hs2 = _tc_mid(A1, hs1, b1r, W2, deg)
    A2 = _sc_aggregate(hs2, row, col, zeros)
    return _tc_final(A2, hs2, b2r, deg)
